# baseline (speedup 1.0000x reference)
import jax
import jax.numpy as jnp
from jax.experimental import pallas as pl
from jax.experimental.pallas import tpu as pltpu

BLOCK_ROWS = 400


def _fused_gcn_kernel(x_ref, w_ref, b_ref, adj_ref, o_ref, z_ref):
    @pl.when(pl.program_id(0) == 0)
    def _compute_z():
        z_ref[...] = jax.lax.dot_general(
            x_ref[...], w_ref[...],
            dimension_numbers=(((1,), (1,)), ((), ())),
            preferred_element_type=jnp.float32,
        )

    y = jnp.dot(adj_ref[...], z_ref[...],
                preferred_element_type=jnp.float32) + b_ref[...]
    o_ref[...] = jnp.maximum(y, 0.0)


@jax.jit
def _run(x, adj, w, b):
    n, d_in = x.shape
    d_out = w.shape[0]
    bm = BLOCK_ROWS
    assert n % bm == 0
    grid = (n // bm,)
    return pl.pallas_call(
        _fused_gcn_kernel,
        grid=grid,
        in_specs=[
            pl.BlockSpec((n, d_in), lambda i: (0, 0)),
            pl.BlockSpec((d_out, d_in), lambda i: (0, 0)),
            pl.BlockSpec((d_out,), lambda i: (0,)),
            pl.BlockSpec((bm, n), lambda i: (i, 0)),
        ],
        out_specs=pl.BlockSpec((bm, d_out), lambda i: (i, 0)),
        out_shape=jax.ShapeDtypeStruct((n, d_out), jnp.float32),
        scratch_shapes=[pltpu.VMEM((n, d_out), jnp.float32)],
        compiler_params=pltpu.CompilerParams(
            dimension_semantics=("arbitrary",),
        ),
    )(x, w, b, adj)


def kernel(input, adj, W, b):
    return _run(input, adj, W, b)
